# baseline (speedup 1.0000x reference)
import jax
import jax.numpy as jnp
from jax.experimental import pallas as pl
from jax.experimental.pallas import tpu as pltpu

TC = 512


def _mm(a, b):
    return jnp.dot(a, b, preferred_element_type=jnp.float32)


def _dot_nn(a, b):
    return jax.lax.dot_general(
        a, b, dimension_numbers=(((1,), (1,)), ((), ())),
        preferred_element_type=jnp.float32)


def _make_kernel(n):
  def _fused_kernel(bt_ref, x0_ref, w1_ref, w2_ref, x0_out_ref, x1_out_ref,
                    bbf_ref, acct_ref, ht_ref, x0bt_ref):
    p = pl.program_id(0)
    j = pl.program_id(1)
    nt = pl.num_programs(1)
    e, tc = bt_ref.shape
    bf16 = jnp.bfloat16

    @pl.when(jnp.logical_and(p == 0, j == 0))
    def _():
        acct_ref[...] = jnp.zeros_like(acct_ref)

    @pl.when(p == 0)
    def _():
        b = bt_ref[...].astype(bf16)

        @pl.when(j == nt - 1)
        def _():
            rem = n - (nt - 1) * tc
            col = jax.lax.broadcasted_iota(jnp.int32, (e, tc), 1)
            bbf_ref[j] = jnp.where(col < rem, b, jnp.zeros_like(b))

        @pl.when(j != nt - 1)
        def _():
            bbf_ref[j] = b

        x0t = x0_ref[...].astype(bf16)
        rem = n - (nt - 1) * tc
        row = jax.lax.broadcasted_iota(jnp.int32, x0t.shape, 0)
        x0t = jnp.where(jnp.logical_or(j != nt - 1, row < rem), x0t,
                        jnp.zeros_like(x0t))
        x0tt = jnp.swapaxes(x0t, 0, 1)
        acct_ref[...] += _dot_nn(x0tt, bbf_ref[j])

    @pl.when(jnp.logical_and(p == 0, j == nt - 1))
    def _():
        w1t = jnp.swapaxes(w1_ref[...], 0, 1).astype(bf16)
        ht_ref[...] = _mm(w1t, acct_ref[...].astype(bf16)).astype(bf16)
        acct_ref[...] = jnp.zeros_like(acct_ref)

    @pl.when(p == 1)
    def _():
        x0bt_ref[:, pl.ds(j * tc, tc)] = _mm(
            ht_ref[...], bbf_ref[j]).astype(bf16)

    @pl.when(p == 2)
    def _():
        acct_ref[...] += _dot_nn(x0bt_ref[:, pl.ds(j * tc, tc)], bbf_ref[j])

    @pl.when(jnp.logical_and(p == 2, j == nt - 1))
    def _():
        x1_out_ref[...] = jnp.swapaxes(acct_ref[...], 0, 1)
        w2t = jnp.swapaxes(w2_ref[...], 0, 1).astype(bf16)
        ht_ref[...] = _mm(w2t, acct_ref[...].astype(bf16)).astype(bf16)

    @pl.when(p == 3)
    def _():
        t = _mm(ht_ref[...], bbf_ref[j])
        x0_out_ref[...] = jnp.swapaxes(t, 0, 1)

  return _fused_kernel


@jax.jit
def kernel(x_0, incidence_1, W1, W2):
    n, e = incidence_1.shape
    d = x_0.shape[1]
    nt = (n + TC - 1) // TC
    f32 = jnp.float32
    bt = incidence_1.T

    x0_out, x1_out = pl.pallas_call(
        _make_kernel(n),
        grid=(4, nt),
        in_specs=[
            pl.BlockSpec((e, TC), lambda p, j: (0, jnp.where(p == 0, j, 0))),
            pl.BlockSpec((TC, d), lambda p, j: (jnp.where(p == 0, j, 0), 0)),
            pl.BlockSpec((d, d), lambda p, j: (0, 0)),
            pl.BlockSpec((d, d), lambda p, j: (0, 0)),
        ],
        out_specs=[
            pl.BlockSpec((TC, d), lambda p, j: (jnp.where(p == 3, j, 0), 0)),
            pl.BlockSpec((e, d), lambda p, j: (0, 0)),
        ],
        out_shape=[
            jax.ShapeDtypeStruct((n, d), f32),
            jax.ShapeDtypeStruct((e, d), f32),
        ],
        scratch_shapes=[
            pltpu.VMEM((nt, e, TC), jnp.bfloat16),
            pltpu.VMEM((d, e), f32),
            pltpu.VMEM((d, e), jnp.bfloat16),
            pltpu.VMEM((d, nt * TC), jnp.bfloat16),
        ],
        compiler_params=pltpu.CompilerParams(
            dimension_semantics=("arbitrary", "arbitrary")),
    )(bt, x_0, W1, W2)

    return (x0_out, x1_out)


# speedup vs baseline: 1.1281x; 1.1281x over previous
import jax
import jax.numpy as jnp
from jax.experimental import pallas as pl
from jax.experimental.pallas import tpu as pltpu

TC = 640


def _mm(a, b):
    return jnp.dot(a, b, preferred_element_type=jnp.float32)


def _dot_nn(a, b):
    return jax.lax.dot_general(
        a, b, dimension_numbers=(((1,), (1,)), ((), ())),
        preferred_element_type=jnp.float32)


def _make_kernel(n):
  def _fused_kernel(bt_ref, x0_ref, w1_ref, w2_ref, x0_out_ref, x1_out_ref,
                    bbf_ref, acct_ref, ht_ref):
    p = pl.program_id(0)
    j = pl.program_id(1)
    nt = pl.num_programs(1)
    e, tc = bt_ref.shape
    bf16 = jnp.bfloat16

    @pl.when(jnp.logical_and(p == 0, j == 0))
    def _():
        acct_ref[...] = jnp.zeros_like(acct_ref)

    @pl.when(p == 0)
    def _():
        b = bt_ref[...].astype(bf16)

        @pl.when(j == nt - 1)
        def _():
            rem = n - (nt - 1) * tc
            col = jax.lax.broadcasted_iota(jnp.int32, (e, tc), 1)
            bbf_ref[j] = jnp.where(col < rem, b, jnp.zeros_like(b))

        @pl.when(j != nt - 1)
        def _():
            bbf_ref[j] = b

        x0t = x0_ref[...].astype(bf16)
        rem = n - (nt - 1) * tc
        row = jax.lax.broadcasted_iota(jnp.int32, x0t.shape, 0)
        x0t = jnp.where(jnp.logical_or(j != nt - 1, row < rem), x0t,
                        jnp.zeros_like(x0t))
        x0tt = jnp.swapaxes(x0t, 0, 1)
        acct_ref[...] += _dot_nn(x0tt, bt_ref[...])

    @pl.when(jnp.logical_and(p == 0, j == nt - 1))
    def _():
        w1t = jnp.swapaxes(w1_ref[...], 0, 1).astype(bf16)
        ht_ref[...] = _mm(w1t, acct_ref[...].astype(bf16)).astype(bf16)
        acct_ref[...] = jnp.zeros_like(acct_ref)

    @pl.when(p == 1)
    def _():
        bb = bbf_ref[j]
        x0bt = _mm(ht_ref[...], bb).astype(bf16)
        acct_ref[...] += _dot_nn(x0bt, bb)

    @pl.when(jnp.logical_and(p == 1, j == nt - 1))
    def _():
        x1_out_ref[...] = jnp.swapaxes(acct_ref[...], 0, 1)
        w2t = jnp.swapaxes(w2_ref[...], 0, 1).astype(bf16)
        ht_ref[...] = _mm(w2t, acct_ref[...].astype(bf16)).astype(bf16)

    @pl.when(p == 2)
    def _():
        t = _mm(ht_ref[...], bbf_ref[j])
        x0_out_ref[...] = jnp.swapaxes(t, 0, 1)

  return _fused_kernel


@jax.jit
def kernel(x_0, incidence_1, W1, W2):
    n, e = incidence_1.shape
    d = x_0.shape[1]
    nt = (n + TC - 1) // TC
    f32 = jnp.float32
    bt = incidence_1.T

    x0_out, x1_out = pl.pallas_call(
        _make_kernel(n),
        grid=(3, nt),
        in_specs=[
            pl.BlockSpec((e, TC), lambda p, j: (0, jnp.where(p == 0, j, 0))),
            pl.BlockSpec((TC, d), lambda p, j: (jnp.where(p == 0, j, 0), 0)),
            pl.BlockSpec((d, d), lambda p, j: (0, 0)),
            pl.BlockSpec((d, d), lambda p, j: (0, 0)),
        ],
        out_specs=[
            pl.BlockSpec((TC, d), lambda p, j: (jnp.where(p == 2, j, 0), 0)),
            pl.BlockSpec((e, d), lambda p, j: (0, 0)),
        ],
        out_shape=[
            jax.ShapeDtypeStruct((n, d), f32),
            jax.ShapeDtypeStruct((e, d), f32),
        ],
        scratch_shapes=[
            pltpu.VMEM((nt, e, TC), jnp.bfloat16),
            pltpu.VMEM((d, e), f32),
            pltpu.VMEM((d, e), jnp.bfloat16),
        ],
        compiler_params=pltpu.CompilerParams(
            dimension_semantics=("arbitrary", "arbitrary")),
    )(bt, x_0, W1, W2)

    return (x0_out, x1_out)
